# speedup vs baseline: 1.9211x; 1.9211x over previous
import functools

import jax
import jax.numpy as jnp
from jax import lax
from jax.experimental import pallas as pl
from jax.experimental.pallas import tpu as pltpu
from jax.experimental.pallas import tpu_sc as plsc

N_NODES = 100000
D = 512
B = 50000

NC = 2
NS = 16
NW = NC * NS
BPW = 1568
BPAD = NW * BPW
CH = 112
G = BPW // CH
TAIL = B - (NW - 1) * BPW - (G - 2) * CH

_mesh = plsc.VectorSubcoreMesh(core_axis_name="c", subcore_axis_name="s")


@functools.partial(
    pl.kernel,
    mesh=_mesh,
    out_type=jax.ShapeDtypeStruct((B, D), jnp.float32),
    scratch_types=[
        pltpu.VMEM((BPW,), jnp.int32),
        pltpu.VMEM((CH, D), jnp.float32),
        pltpu.VMEM((CH, D), jnp.float32),
        pltpu.SemaphoreType.DMA,
        pltpu.SemaphoreType.DMA,
        pltpu.SemaphoreType.DMA,
        pltpu.SemaphoreType.DMA,
    ],
)
def _sc_gather(nodes_hbm, table_hbm, out_hbm, idx_v, buf0, buf1,
               gsem0, gsem1, wsem0, wsem1):
    wid = lax.axis_index("s") * NC + lax.axis_index("c")
    base = wid * BPW
    is_last = wid == NW - 1

    pltpu.sync_copy(nodes_hbm.at[pl.ds(base, BPW)], idx_v)

    bufs = (buf0, buf1)
    gsems = (gsem0, gsem1)
    wsems = (wsem0, wsem1)

    def start_gather(g):
        b = g % 2
        return pltpu.async_copy(
            table_hbm.at[idx_v.at[pl.ds(g * CH, CH)]], bufs[b], gsems[b])

    whandles = [None, None]
    h = start_gather(0)
    for g in range(G):
        b = g % 2
        nh = None
        if g + 1 < G:
            if whandles[1 - b] is not None:
                whandles[1 - b].wait()
                whandles[1 - b] = None
            nh = start_gather(g + 1)
        h.wait()
        row0 = base + g * CH
        if g < G - 2:
            whandles[b] = pltpu.async_copy(
                bufs[b], out_hbm.at[pl.ds(row0, CH)], wsems[b])
        elif g == G - 2:
            @pl.when(jnp.logical_not(is_last))
            def _():
                pltpu.sync_copy(bufs[b], out_hbm.at[pl.ds(row0, CH)])

            @pl.when(is_last)
            def _():
                pltpu.sync_copy(bufs[b].at[pl.ds(0, TAIL)],
                                out_hbm.at[pl.ds(B - TAIL, TAIL)])
        else:
            @pl.when(jnp.logical_not(is_last))
            def _():
                pltpu.sync_copy(bufs[b], out_hbm.at[pl.ds(row0, CH)])
        h = nh
    for wh in whandles:
        if wh is not None:
            wh.wait()


def kernel(nodes, table):
    nodes = nodes.astype(jnp.int32)
    nodes_pad = jnp.concatenate(
        [nodes, jnp.zeros((BPAD - B,), jnp.int32)])
    return _sc_gather(nodes_pad, table)


# speedup vs baseline: 1.9336x; 1.0065x over previous
import functools

import jax
import jax.numpy as jnp
from jax import lax
from jax.experimental import pallas as pl
from jax.experimental.pallas import tpu as pltpu
from jax.experimental.pallas import tpu_sc as plsc

N_NODES = 100000
D = 512
B = 50000

NC = 2
NS = 16
NW = NC * NS
BPW = 1568
CH = 112
G = BPW // CH
LAST_BASE = (NW - 1) * BPW
LAST_N = B - LAST_BASE
TAIL = LAST_N - (G - 2) * CH

_mesh = plsc.VectorSubcoreMesh(core_axis_name="c", subcore_axis_name="s")


@functools.partial(
    pl.kernel,
    mesh=_mesh,
    out_type=jax.ShapeDtypeStruct((B, D), jnp.float32),
    scratch_types=[
        pltpu.VMEM((BPW,), jnp.int32),
        pltpu.VMEM((CH, D), jnp.float32),
        pltpu.VMEM((CH, D), jnp.float32),
        pltpu.SemaphoreType.DMA,
        pltpu.SemaphoreType.DMA,
        pltpu.SemaphoreType.DMA,
        pltpu.SemaphoreType.DMA,
    ],
)
def _sc_gather(nodes_hbm, table_hbm, out_hbm, idx_v, buf0, buf1,
               gsem0, gsem1, wsem0, wsem1):
    wid = lax.axis_index("s") * NC + lax.axis_index("c")
    base = wid * BPW
    is_last = wid == NW - 1

    @pl.when(jnp.logical_not(is_last))
    def _():
        pltpu.sync_copy(nodes_hbm.at[pl.ds(base, BPW)], idx_v)

    @pl.when(is_last)
    def _():
        pltpu.sync_copy(nodes_hbm.at[pl.ds(LAST_BASE, LAST_N)],
                        idx_v.at[pl.ds(0, LAST_N)])
        zeros = jnp.zeros((16,), jnp.int32)
        for i in range(LAST_N, BPW, 16):
            idx_v[pl.ds(i, 16)] = zeros

    bufs = (buf0, buf1)
    gsems = (gsem0, gsem1)
    wsems = (wsem0, wsem1)

    def start_gather(g):
        b = g % 2
        return pltpu.async_copy(
            table_hbm.at[idx_v.at[pl.ds(g * CH, CH)]], bufs[b], gsems[b])

    whandles = [None, None]
    h = start_gather(0)
    for g in range(G):
        b = g % 2
        nh = None
        if g + 1 < G:
            if whandles[1 - b] is not None:
                whandles[1 - b].wait()
                whandles[1 - b] = None
            nh = start_gather(g + 1)
        h.wait()
        row0 = base + g * CH
        if g < G - 2:
            whandles[b] = pltpu.async_copy(
                bufs[b], out_hbm.at[pl.ds(row0, CH)], wsems[b])
        elif g == G - 2:
            @pl.when(jnp.logical_not(is_last))
            def _():
                pltpu.sync_copy(bufs[b], out_hbm.at[pl.ds(row0, CH)])

            @pl.when(is_last)
            def _():
                pltpu.sync_copy(bufs[b].at[pl.ds(0, TAIL)],
                                out_hbm.at[pl.ds(B - TAIL, TAIL)])
        else:
            @pl.when(jnp.logical_not(is_last))
            def _():
                pltpu.sync_copy(bufs[b], out_hbm.at[pl.ds(row0, CH)])
        h = nh
    for wh in whandles:
        if wh is not None:
            wh.wait()


def kernel(nodes, table):
    return _sc_gather(nodes.astype(jnp.int32), table)


# speedup vs baseline: 1.9421x; 1.0044x over previous
import functools

import jax
import jax.numpy as jnp
from jax import lax
from jax.experimental import pallas as pl
from jax.experimental.pallas import tpu as pltpu
from jax.experimental.pallas import tpu_sc as plsc

N_NODES = 100000
D = 512
B = 50000

NC = 2
NS = 16
NW = NC * NS
BPW = 1568
CH = 112
G = BPW // CH
LAST_BASE = (NW - 1) * BPW
LAST_N = B - LAST_BASE
TAIL = LAST_N - (G - 2) * CH

_mesh = plsc.VectorSubcoreMesh(core_axis_name="c", subcore_axis_name="s")


@functools.partial(
    pl.kernel,
    mesh=_mesh,
    out_type=jax.ShapeDtypeStruct((B, D), jnp.float32),
    scratch_types=[
        pltpu.VMEM((BPW,), jnp.int32),
        pltpu.VMEM((CH, D), jnp.float32),
        pltpu.VMEM((CH, D), jnp.float32),
        pltpu.SemaphoreType.DMA,
        pltpu.SemaphoreType.DMA,
        pltpu.SemaphoreType.DMA,
        pltpu.SemaphoreType.DMA,
    ],
)
def _sc_gather(nodes_hbm, table_hbm, out_hbm, idx_v, buf0, buf1,
               gsem0, gsem1, wsem0, wsem1):
    wid = lax.axis_index("s") * NC + lax.axis_index("c")
    base = wid * BPW
    is_last = wid == NW - 1

    @pl.when(jnp.logical_not(is_last))
    def _():
        pltpu.sync_copy(nodes_hbm.at[pl.ds(base, BPW)], idx_v)

    @pl.when(is_last)
    def _():
        pltpu.sync_copy(nodes_hbm.at[pl.ds(LAST_BASE, LAST_N)],
                        idx_v.at[pl.ds(0, LAST_N)])
        zeros = jnp.zeros((16,), jnp.int32)
        for i in range(LAST_N, BPW, 16):
            idx_v[pl.ds(i, 16)] = zeros

    bufs = (buf0, buf1)
    gsems = (gsem0, gsem1)
    wsems = (wsem0, wsem1)

    def start_gather(g):
        b = g % 2
        return pltpu.async_copy(
            table_hbm.at[idx_v.at[pl.ds(g * CH, CH)]], bufs[b], gsems[b])

    whandles = [None, None]
    h = start_gather(0)
    for g in range(G):
        b = g % 2
        nh = None
        if g + 1 < G:
            if whandles[1 - b] is not None:
                whandles[1 - b].wait()
                whandles[1 - b] = None
            nh = start_gather(g + 1)
        h.wait()
        row0 = base + g * CH
        if g < G - 2:
            whandles[b] = pltpu.async_copy(
                bufs[b], out_hbm.at[pl.ds(row0, CH)], wsems[b])
        elif g == G - 2:
            @pl.when(jnp.logical_not(is_last))
            def _():
                pltpu.async_copy(bufs[b], out_hbm.at[pl.ds(row0, CH)],
                                 wsems[b])

            @pl.when(is_last)
            def _():
                pltpu.async_copy(bufs[b].at[pl.ds(0, TAIL)],
                                 out_hbm.at[pl.ds(B - TAIL, TAIL)], wsems[b])
        else:
            @pl.when(jnp.logical_not(is_last))
            def _():
                pltpu.async_copy(bufs[b], out_hbm.at[pl.ds(row0, CH)],
                                 wsems[b])
        h = nh
    for wh in whandles:
        if wh is not None:
            wh.wait()

    b2 = (G - 2) % 2
    b1 = (G - 1) % 2
    row2 = base + (G - 2) * CH
    row1 = base + (G - 1) * CH

    @pl.when(jnp.logical_not(is_last))
    def _():
        pltpu.make_async_copy(bufs[b2], out_hbm.at[pl.ds(row2, CH)],
                              wsems[b2]).wait()
        pltpu.make_async_copy(bufs[b1], out_hbm.at[pl.ds(row1, CH)],
                              wsems[b1]).wait()

    @pl.when(is_last)
    def _():
        pltpu.make_async_copy(bufs[b2].at[pl.ds(0, TAIL)],
                              out_hbm.at[pl.ds(B - TAIL, TAIL)],
                              wsems[b2]).wait()


def kernel(nodes, table):
    return _sc_gather(nodes.astype(jnp.int32), table)
